# baseline (speedup 1.0000x reference)
import functools

import jax
import jax.numpy as jnp
from jax import lax
from jax.experimental import pallas as pl
from jax.experimental.pallas import tpu as pltpu
from jax.experimental.pallas import tpu_sc as plsc

N_NODES, N_EDGES, DIM = 10000, 320000, 128

NC, NS = 2, 16
NW = NC * NS
CHUNK = 80
EPT = 10400
E_PAD = NW * EPT
NCHUNK = EPT // CHUNK
N_PAD = 10240
RPT = N_PAD // NS
BLK = 1000


def _relu_body(x_ref, o_ref):
    o_ref[...] = jnp.maximum(x_ref[...], 0.0)


def _mlp_body(x_ref, a0_ref, a1_ref, w1_ref, b1_ref, w2_ref, b2_ref,
              eps_ref, o_ref):
    h = x_ref[...] * (1.0 + eps_ref[0, 0]) + a0_ref[...] + a1_ref[...]
    h = jnp.dot(h, w1_ref[...], preferred_element_type=jnp.float32)
    h = jnp.maximum(h + b1_ref[...], 0.0)
    o = jnp.dot(h, w2_ref[...], preferred_element_type=jnp.float32)
    o_ref[...] = o + b2_ref[...]


IB = 5
RB = 2
UNROLL = 10
NROUND = NCHUNK // UNROLL


def _sc_agg_body(xr_hbm, src_hbm, dst_hbm, zeros_hbm, agg_hbm, *rest):
    sidx = rest[0:IB]
    didx = rest[IB:2 * IB]
    rows = rest[2 * IB:2 * IB + RB]
    isem = rest[2 * IB + RB:3 * IB + RB]
    gsem = rest[3 * IB + RB:3 * IB + 2 * RB]
    acc = rest[3 * IB + 2 * RB]

    c = lax.axis_index("c")
    s = lax.axis_index("s")
    wid = s * NC + c
    pltpu.sync_copy(zeros_hbm, acc.at[pl.ds(s * RPT, RPT)])
    plsc.subcore_barrier()

    base = wid * EPT
    last_off = base + (NCHUNK - 1) * CHUNK

    def idx_start(off, b):
        pltpu.make_async_copy(src_hbm.at[pl.ds(off, CHUNK)], sidx[b],
                              isem[b]).start()
        pltpu.make_async_copy(dst_hbm.at[pl.ds(off, CHUNK)], didx[b],
                              isem[b]).start()

    def idx_wait(b):
        pltpu.make_async_copy(src_hbm.at[pl.ds(base, CHUNK)], sidx[b],
                              isem[b]).wait()
        pltpu.make_async_copy(dst_hbm.at[pl.ds(base, CHUNK)], didx[b],
                              isem[b]).wait()

    def g_start(b, rb):
        pltpu.make_async_copy(xr_hbm.at[sidx[b]], rows[rb], gsem[rb]).start()

    def g_wait(b, rb):
        pltpu.make_async_copy(xr_hbm.at[sidx[b]], rows[rb], gsem[rb]).wait()

    for j in range(IB):
        idx_start(base + j * CHUNK, j)
    idx_wait(0)
    g_start(0, 0)

    def round_body(r, carry):
        k0 = r * UNROLL
        for u in range(UNROLL):
            k = k0 + u
            b = u % IB
            b1 = (u + 1) % IB
            rb = u % RB
            rb1 = (u + 1) % RB
            g_wait(b, rb)
            if RB > 1:
                idx_wait(b1)
                g_start(b1, rb1)
            pltpu.sync_copy(rows[rb], acc.at[didx[b]], add=True)
            if RB == 1:
                idx_wait(b1)
                g_start(b1, rb1)
            off = jnp.minimum(base + (k + IB) * CHUNK, last_off)
            idx_start(off, b)
        return carry

    lax.fori_loop(0, NROUND, round_body, 0)

    g_wait(0, 0)
    for b in range(1, IB):
        idx_wait(b)

    plsc.subcore_barrier()
    pltpu.sync_copy(acc.at[pl.ds(s * RPT, RPT)],
                    agg_hbm.at[pl.ds(c * N_PAD + s * RPT, RPT)])


def kernel(x, edge_index, W1, b1, W2, b2, eps):
    npad = E_PAD - N_EDGES
    src = jnp.concatenate([edge_index[0],
                           jnp.zeros((npad,), dtype=jnp.int32)])
    dst = jnp.concatenate([edge_index[1],
                           jnp.full((npad,), N_PAD - 1, dtype=jnp.int32)])

    xr = pl.pallas_call(
        _relu_body,
        grid=(N_NODES // BLK,),
        in_specs=[pl.BlockSpec((BLK, DIM), lambda i: (i, 0))],
        out_specs=pl.BlockSpec((BLK, DIM), lambda i: (i, 0)),
        out_shape=jax.ShapeDtypeStruct((N_NODES, DIM), jnp.float32),
    )(x)

    agg_fn = pl.kernel(
        _sc_agg_body,
        out_type=jax.ShapeDtypeStruct((NC * N_PAD, DIM), jnp.float32),
        mesh=plsc.VectorSubcoreMesh(core_axis_name="c", subcore_axis_name="s"),
        scratch_types=(
            [pltpu.VMEM((CHUNK,), jnp.int32)] * IB
            + [pltpu.VMEM((CHUNK,), jnp.int32)] * IB
            + [pltpu.VMEM((CHUNK, DIM), jnp.float32)] * RB
            + [pltpu.SemaphoreType.DMA] * IB
            + [pltpu.SemaphoreType.DMA] * RB
            + [pltpu.VMEM_SHARED((N_PAD, DIM), jnp.float32)]
        ),
    )
    aggp = agg_fn(xr, src, dst, jnp.zeros((RPT, DIM), jnp.float32))
    agg0 = aggp[:N_NODES]
    agg1 = aggp[N_PAD:N_PAD + N_NODES]

    out = pl.pallas_call(
        _mlp_body,
        grid=(N_NODES // BLK,),
        in_specs=[
            pl.BlockSpec((BLK, DIM), lambda i: (i, 0)),
            pl.BlockSpec((BLK, DIM), lambda i: (i, 0)),
            pl.BlockSpec((BLK, DIM), lambda i: (i, 0)),
            pl.BlockSpec((DIM, DIM), lambda i: (0, 0)),
            pl.BlockSpec((1, DIM), lambda i: (0, 0)),
            pl.BlockSpec((DIM, DIM), lambda i: (0, 0)),
            pl.BlockSpec((1, DIM), lambda i: (0, 0)),
            pl.BlockSpec((1, 1), lambda i: (0, 0), memory_space=pltpu.SMEM),
        ],
        out_specs=pl.BlockSpec((BLK, DIM), lambda i: (i, 0)),
        out_shape=jax.ShapeDtypeStruct((N_NODES, DIM), jnp.float32),
    )(x, agg0, agg1, W1, b1.reshape(1, DIM), W2, b2.reshape(1, DIM),
      eps.reshape(1, 1))
    return out


# speedup vs baseline: 1.4338x; 1.4338x over previous
import functools

import jax
import jax.numpy as jnp
from jax import lax
from jax.experimental import pallas as pl
from jax.experimental.pallas import tpu as pltpu
from jax.experimental.pallas import tpu_sc as plsc

N_NODES, N_EDGES, DIM = 10000, 320000, 128

NC, NS = 2, 16
NW = NC * NS
CHUNK = 128
EPT = 10240
E_PAD = NW * EPT
NCHUNK = EPT // CHUNK
N_PAD = 10240
RPT = N_PAD // NS
BLK = 1000


def _relu_body(x_ref, o_ref):
    o_ref[...] = jnp.maximum(x_ref[...], 0.0)


def _mlp_body(x_ref, a0_ref, a1_ref, w1_ref, b1_ref, w2_ref, b2_ref,
              eps_ref, o_ref):
    h = x_ref[...] * (1.0 + eps_ref[0, 0]) + a0_ref[...] + a1_ref[...]
    h = jnp.dot(h, w1_ref[...], preferred_element_type=jnp.float32)
    h = jnp.maximum(h + b1_ref[...], 0.0)
    o = jnp.dot(h, w2_ref[...], preferred_element_type=jnp.float32)
    o_ref[...] = o + b2_ref[...]


IB = 2
RB = 2
UNROLL = 2
NROUND = NCHUNK // UNROLL


def _sc_agg_body(xr_hbm, src_hbm, dst_hbm, zeros_hbm, agg_hbm, *rest):
    sidx = rest[0:IB]
    didx = rest[IB:2 * IB]
    rows = rest[2 * IB:2 * IB + RB]
    isem = rest[2 * IB + RB:3 * IB + RB]
    gsem = rest[3 * IB + RB:3 * IB + 2 * RB]
    acc = rest[3 * IB + 2 * RB]

    c = lax.axis_index("c")
    s = lax.axis_index("s")
    wid = s * NC + c
    pltpu.sync_copy(zeros_hbm, acc.at[pl.ds(s * RPT, RPT)])
    plsc.subcore_barrier()

    base = wid * EPT
    last_off = base + (NCHUNK - 1) * CHUNK

    def idx_start(off, b):
        pltpu.make_async_copy(src_hbm.at[pl.ds(off, CHUNK)], sidx[b],
                              isem[b]).start()
        pltpu.make_async_copy(dst_hbm.at[pl.ds(off, CHUNK)], didx[b],
                              isem[b]).start()

    def idx_wait(b):
        pltpu.make_async_copy(src_hbm.at[pl.ds(base, CHUNK)], sidx[b],
                              isem[b]).wait()
        pltpu.make_async_copy(dst_hbm.at[pl.ds(base, CHUNK)], didx[b],
                              isem[b]).wait()

    def g_start(b, rb):
        pltpu.make_async_copy(xr_hbm.at[sidx[b]], rows[rb], gsem[rb]).start()

    def g_wait(b, rb):
        pltpu.make_async_copy(xr_hbm.at[sidx[b]], rows[rb], gsem[rb]).wait()

    for j in range(IB):
        idx_start(base + j * CHUNK, j)
    idx_wait(0)
    g_start(0, 0)

    def round_body(r, carry):
        k0 = r * UNROLL
        for u in range(UNROLL):
            k = k0 + u
            b = u % IB
            b1 = (u + 1) % IB
            rb = u % RB
            rb1 = (u + 1) % RB
            g_wait(b, rb)
            if RB > 1:
                idx_wait(b1)
                g_start(b1, rb1)
            pltpu.sync_copy(rows[rb], acc.at[didx[b]], add=True)
            if RB == 1:
                idx_wait(b1)
                g_start(b1, rb1)
            off = jnp.minimum(base + (k + IB) * CHUNK, last_off)
            idx_start(off, b)
        return carry

    lax.fori_loop(0, NROUND, round_body, 0)

    g_wait(0, 0)
    for b in range(1, IB):
        idx_wait(b)

    plsc.subcore_barrier()
    pltpu.sync_copy(acc.at[pl.ds(s * RPT, RPT)],
                    agg_hbm.at[pl.ds(c * N_PAD + s * RPT, RPT)])


def kernel(x, edge_index, W1, b1, W2, b2, eps):
    npad = E_PAD - N_EDGES
    src = jnp.concatenate([edge_index[0],
                           jnp.zeros((npad,), dtype=jnp.int32)])
    dst = jnp.concatenate([edge_index[1],
                           jnp.full((npad,), N_PAD - 1, dtype=jnp.int32)])

    xr = pl.pallas_call(
        _relu_body,
        grid=(N_NODES // BLK,),
        in_specs=[pl.BlockSpec((BLK, DIM), lambda i: (i, 0))],
        out_specs=pl.BlockSpec((BLK, DIM), lambda i: (i, 0)),
        out_shape=jax.ShapeDtypeStruct((N_NODES, DIM), jnp.float32),
    )(x)

    agg_fn = pl.kernel(
        _sc_agg_body,
        out_type=jax.ShapeDtypeStruct((NC * N_PAD, DIM), jnp.float32),
        mesh=plsc.VectorSubcoreMesh(core_axis_name="c", subcore_axis_name="s"),
        scratch_types=(
            [pltpu.VMEM((CHUNK,), jnp.int32)] * IB
            + [pltpu.VMEM((CHUNK,), jnp.int32)] * IB
            + [pltpu.VMEM((CHUNK, DIM), jnp.float32)] * RB
            + [pltpu.SemaphoreType.DMA] * IB
            + [pltpu.SemaphoreType.DMA] * RB
            + [pltpu.VMEM_SHARED((N_PAD, DIM), jnp.float32)]
        ),
    )
    aggp = agg_fn(xr, src, dst, jnp.zeros((RPT, DIM), jnp.float32))
    agg0 = aggp[:N_NODES]
    agg1 = aggp[N_PAD:N_PAD + N_NODES]

    out = pl.pallas_call(
        _mlp_body,
        grid=(N_NODES // BLK,),
        in_specs=[
            pl.BlockSpec((BLK, DIM), lambda i: (i, 0)),
            pl.BlockSpec((BLK, DIM), lambda i: (i, 0)),
            pl.BlockSpec((BLK, DIM), lambda i: (i, 0)),
            pl.BlockSpec((DIM, DIM), lambda i: (0, 0)),
            pl.BlockSpec((1, DIM), lambda i: (0, 0)),
            pl.BlockSpec((DIM, DIM), lambda i: (0, 0)),
            pl.BlockSpec((1, DIM), lambda i: (0, 0)),
            pl.BlockSpec((1, 1), lambda i: (0, 0), memory_space=pltpu.SMEM),
        ],
        out_specs=pl.BlockSpec((BLK, DIM), lambda i: (i, 0)),
        out_shape=jax.ShapeDtypeStruct((N_NODES, DIM), jnp.float32),
    )(x, agg0, agg1, W1, b1.reshape(1, DIM), W2, b2.reshape(1, DIM),
      eps.reshape(1, 1))
    return out


# speedup vs baseline: 4.0333x; 2.8131x over previous
import functools

import jax
import jax.numpy as jnp
from jax import lax
from jax.experimental import pallas as pl
from jax.experimental.pallas import tpu as pltpu
from jax.experimental.pallas import tpu_sc as plsc

N_NODES, N_EDGES, DIM = 10000, 320000, 128

NC, NS = 2, 16
NW = NC * NS
CHUNK = 128
EPT = 10240
E_PAD = NW * EPT
NCHUNK = EPT // CHUNK
N_PAD = 10240
RPT = N_PAD // NS
BLK = 1000


def _relu_body(x_ref, o_ref):
    o_ref[...] = jnp.maximum(x_ref[...], 0.0)


def _mlp_body(x_ref, a0_ref, a1_ref, w1_ref, b1_ref, w2_ref, b2_ref,
              eps_ref, o_ref):
    h = x_ref[...] * (1.0 + eps_ref[0, 0]) + a0_ref[...] + a1_ref[...]
    h = jnp.dot(h, w1_ref[...], preferred_element_type=jnp.float32)
    h = jnp.maximum(h + b1_ref[...], 0.0)
    o = jnp.dot(h, w2_ref[...], preferred_element_type=jnp.float32)
    o_ref[...] = o + b2_ref[...]


IB = 2
RB = 2
UNROLL = 2
NROUND = NCHUNK // UNROLL


def _sc_agg_body(xr_hbm, src_hbm, dst_hbm, zeros_hbm, agg_hbm, *rest):
    sidx = rest[0:IB]
    didx = rest[IB:2 * IB]
    rows = rest[2 * IB:2 * IB + RB]
    isem = rest[2 * IB + RB:3 * IB + RB]
    gsem = rest[3 * IB + RB:3 * IB + 2 * RB]
    acc = rest[3 * IB + 2 * RB]

    c = lax.axis_index("c")
    s = lax.axis_index("s")
    wid = s * NC + c
    pltpu.sync_copy(zeros_hbm, acc.at[pl.ds(s * RPT, RPT)])
    plsc.subcore_barrier()

    base = wid * EPT
    last_off = base + (NCHUNK - 1) * CHUNK

    def idx_start(off, b):
        pltpu.make_async_copy(src_hbm.at[pl.ds(off, CHUNK)], sidx[b],
                              isem[b]).start()
        pltpu.make_async_copy(dst_hbm.at[pl.ds(off, CHUNK)], didx[b],
                              isem[b]).start()

    def idx_wait(b):
        pltpu.make_async_copy(src_hbm.at[pl.ds(base, CHUNK)], sidx[b],
                              isem[b]).wait()
        pltpu.make_async_copy(dst_hbm.at[pl.ds(base, CHUNK)], didx[b],
                              isem[b]).wait()

    def g_start(b, rb):
        pltpu.make_async_copy(xr_hbm.at[sidx[b]], rows[rb], gsem[rb]).start()

    def g_wait(b, rb):
        pltpu.make_async_copy(xr_hbm.at[sidx[b]], rows[rb], gsem[rb]).wait()

    for j in range(IB):
        idx_start(base + j * CHUNK, j)
    idx_wait(0)
    g_start(0, 0)

    def round_body(r, carry):
        k0 = r * UNROLL
        for u in range(UNROLL):
            k = k0 + u
            b = u % IB
            b1 = (u + 1) % IB
            rb = u % RB
            rb1 = (u + 1) % RB
            g_wait(b, rb)
            if RB > 1:
                idx_wait(b1)
                g_start(b1, rb1)
            pltpu.sync_copy(rows[rb], acc.at[didx[b]], add=True)
            if RB == 1:
                idx_wait(b1)
                g_start(b1, rb1)
            off = jnp.minimum(base + (k + IB) * CHUNK, last_off)
            idx_start(off, b)
        return carry

    lax.fori_loop(0, NROUND, round_body, 0)

    g_wait(0, 0)
    for b in range(1, IB):
        idx_wait(b)

    plsc.subcore_barrier()
    pltpu.sync_copy(acc.at[pl.ds(s * RPT, RPT)],
                    agg_hbm.at[pl.ds(c * N_PAD + s * RPT, RPT)])


def kernel(x, edge_index, W1, b1, W2, b2, eps):
    npad = E_PAD - N_EDGES
    pad_iota = lax.iota(jnp.int32, npad)
    src = jnp.concatenate([edge_index[0], pad_iota % N_NODES])
    dst = jnp.concatenate([edge_index[1],
                           N_NODES + pad_iota % (N_PAD - N_NODES)])

    xr = pl.pallas_call(
        _relu_body,
        grid=(N_NODES // BLK,),
        in_specs=[pl.BlockSpec((BLK, DIM), lambda i: (i, 0))],
        out_specs=pl.BlockSpec((BLK, DIM), lambda i: (i, 0)),
        out_shape=jax.ShapeDtypeStruct((N_NODES, DIM), jnp.float32),
    )(x)

    agg_fn = pl.kernel(
        _sc_agg_body,
        out_type=jax.ShapeDtypeStruct((NC * N_PAD, DIM), jnp.float32),
        mesh=plsc.VectorSubcoreMesh(core_axis_name="c", subcore_axis_name="s"),
        scratch_types=(
            [pltpu.VMEM((CHUNK,), jnp.int32)] * IB
            + [pltpu.VMEM((CHUNK,), jnp.int32)] * IB
            + [pltpu.VMEM((CHUNK, DIM), jnp.float32)] * RB
            + [pltpu.SemaphoreType.DMA] * IB
            + [pltpu.SemaphoreType.DMA] * RB
            + [pltpu.VMEM_SHARED((N_PAD, DIM), jnp.float32)]
        ),
    )
    aggp = agg_fn(xr, src, dst, jnp.zeros((RPT, DIM), jnp.float32))
    agg0 = aggp[:N_NODES]
    agg1 = aggp[N_PAD:N_PAD + N_NODES]

    out = pl.pallas_call(
        _mlp_body,
        grid=(N_NODES // BLK,),
        in_specs=[
            pl.BlockSpec((BLK, DIM), lambda i: (i, 0)),
            pl.BlockSpec((BLK, DIM), lambda i: (i, 0)),
            pl.BlockSpec((BLK, DIM), lambda i: (i, 0)),
            pl.BlockSpec((DIM, DIM), lambda i: (0, 0)),
            pl.BlockSpec((1, DIM), lambda i: (0, 0)),
            pl.BlockSpec((DIM, DIM), lambda i: (0, 0)),
            pl.BlockSpec((1, DIM), lambda i: (0, 0)),
            pl.BlockSpec((1, 1), lambda i: (0, 0), memory_space=pltpu.SMEM),
        ],
        out_specs=pl.BlockSpec((BLK, DIM), lambda i: (i, 0)),
        out_shape=jax.ShapeDtypeStruct((N_NODES, DIM), jnp.float32),
    )(x, agg0, agg1, W1, b1.reshape(1, DIM), W2, b2.reshape(1, DIM),
      eps.reshape(1, 1))
    return out


# speedup vs baseline: 4.3083x; 1.0682x over previous
import functools

import jax
import jax.numpy as jnp
from jax import lax
from jax.experimental import pallas as pl
from jax.experimental.pallas import tpu as pltpu
from jax.experimental.pallas import tpu_sc as plsc

N_NODES, N_EDGES, DIM = 10000, 320000, 128

NC, NS = 2, 16
NW = NC * NS
CHUNK = 128
EPT = 10240
E_PAD = NW * EPT
NCHUNK = EPT // CHUNK
N_PAD = 10240
RPT = N_PAD // NS
BLK = 1000


ER = N_EDGES // 128
PR = E_PAD // 128 - ER


def _relu_body(x_ref, o_ref):
    o_ref[...] = jnp.maximum(x_ref[...], 0.0)


def _pad_body(e_ref, sp_ref, dp_ref):
    sp_ref[0:ER] = e_ref[0]
    dp_ref[0:ER] = e_ref[1]
    g = (lax.broadcasted_iota(jnp.int32, (PR, 128), 0) * 128
         + lax.broadcasted_iota(jnp.int32, (PR, 128), 1))
    sp_ref[ER:ER + PR] = g % N_NODES
    dp_ref[ER:ER + PR] = N_NODES + g % (N_PAD - N_NODES)


def _mlp_body(x_ref, a0_ref, a1_ref, w1_ref, b1_ref, w2_ref, b2_ref,
              eps_ref, o_ref):
    h = x_ref[...] * (1.0 + eps_ref[0, 0]) + a0_ref[0] + a1_ref[0]
    h = jnp.dot(h, w1_ref[...], preferred_element_type=jnp.float32)
    h = jnp.maximum(h + b1_ref[...], 0.0)
    o = jnp.dot(h, w2_ref[...], preferred_element_type=jnp.float32)
    o_ref[...] = o + b2_ref[...]


IB = 2
RB = 2
UNROLL = 2
NROUND = NCHUNK // UNROLL


def _sc_agg_body(xr_hbm, src_hbm, dst_hbm, zeros_hbm, agg_hbm, *rest):
    sidx = rest[0:IB]
    didx = rest[IB:2 * IB]
    rows = rest[2 * IB:2 * IB + RB]
    isem = rest[2 * IB + RB:3 * IB + RB]
    gsem = rest[3 * IB + RB:3 * IB + 2 * RB]
    acc = rest[3 * IB + 2 * RB]

    c = lax.axis_index("c")
    s = lax.axis_index("s")
    wid = s * NC + c
    pltpu.sync_copy(zeros_hbm, acc.at[pl.ds(s * RPT, RPT)])
    plsc.subcore_barrier()

    base = wid * EPT
    last_off = base + (NCHUNK - 1) * CHUNK

    def idx_start(off, b):
        pltpu.make_async_copy(src_hbm.at[pl.ds(off, CHUNK)], sidx[b],
                              isem[b]).start()
        pltpu.make_async_copy(dst_hbm.at[pl.ds(off, CHUNK)], didx[b],
                              isem[b]).start()

    def idx_wait(b):
        pltpu.make_async_copy(src_hbm.at[pl.ds(base, CHUNK)], sidx[b],
                              isem[b]).wait()
        pltpu.make_async_copy(dst_hbm.at[pl.ds(base, CHUNK)], didx[b],
                              isem[b]).wait()

    def g_start(b, rb):
        pltpu.make_async_copy(xr_hbm.at[sidx[b]], rows[rb], gsem[rb]).start()

    def g_wait(b, rb):
        pltpu.make_async_copy(xr_hbm.at[sidx[b]], rows[rb], gsem[rb]).wait()

    for j in range(IB):
        idx_start(base + j * CHUNK, j)
    idx_wait(0)
    g_start(0, 0)

    def round_body(r, carry):
        k0 = r * UNROLL
        for u in range(UNROLL):
            k = k0 + u
            b = u % IB
            b1 = (u + 1) % IB
            rb = u % RB
            rb1 = (u + 1) % RB
            g_wait(b, rb)
            if RB > 1:
                idx_wait(b1)
                g_start(b1, rb1)
            pltpu.sync_copy(rows[rb], acc.at[didx[b]], add=True)
            if RB == 1:
                idx_wait(b1)
                g_start(b1, rb1)
            off = jnp.minimum(base + (k + IB) * CHUNK, last_off)
            idx_start(off, b)
        return carry

    lax.fori_loop(0, NROUND, round_body, 0)

    g_wait(0, 0)
    for b in range(1, IB):
        idx_wait(b)

    plsc.subcore_barrier()
    pltpu.sync_copy(acc.at[pl.ds(s * RPT, RPT)],
                    agg_hbm.at[pl.ds(c * N_PAD + s * RPT, RPT)])


def kernel(x, edge_index, W1, b1, W2, b2, eps):
    edge3d = edge_index.reshape(2, ER, 128)
    sp, dp = pl.pallas_call(
        _pad_body,
        out_shape=[
            jax.ShapeDtypeStruct((ER + PR, 128), jnp.int32),
            jax.ShapeDtypeStruct((ER + PR, 128), jnp.int32),
        ],
    )(edge3d)
    src = sp.reshape(E_PAD)
    dst = dp.reshape(E_PAD)

    xr = pl.pallas_call(
        _relu_body,
        grid=(N_NODES // BLK,),
        in_specs=[pl.BlockSpec((BLK, DIM), lambda i: (i, 0))],
        out_specs=pl.BlockSpec((BLK, DIM), lambda i: (i, 0)),
        out_shape=jax.ShapeDtypeStruct((N_NODES, DIM), jnp.float32),
    )(x)

    agg_fn = pl.kernel(
        _sc_agg_body,
        out_type=jax.ShapeDtypeStruct((NC * N_PAD, DIM), jnp.float32),
        mesh=plsc.VectorSubcoreMesh(core_axis_name="c", subcore_axis_name="s"),
        scratch_types=(
            [pltpu.VMEM((CHUNK,), jnp.int32)] * IB
            + [pltpu.VMEM((CHUNK,), jnp.int32)] * IB
            + [pltpu.VMEM((CHUNK, DIM), jnp.float32)] * RB
            + [pltpu.SemaphoreType.DMA] * IB
            + [pltpu.SemaphoreType.DMA] * RB
            + [pltpu.VMEM_SHARED((N_PAD, DIM), jnp.float32)]
        ),
    )
    aggp = agg_fn(xr, src, dst, jnp.zeros((RPT, DIM), jnp.float32))
    agg3d = aggp.reshape(NC, N_PAD, DIM)

    out = pl.pallas_call(
        _mlp_body,
        grid=(N_NODES // BLK,),
        in_specs=[
            pl.BlockSpec((BLK, DIM), lambda i: (i, 0)),
            pl.BlockSpec((1, BLK, DIM), lambda i: (0, i, 0)),
            pl.BlockSpec((1, BLK, DIM), lambda i: (1, i, 0)),
            pl.BlockSpec((DIM, DIM), lambda i: (0, 0)),
            pl.BlockSpec((1, DIM), lambda i: (0, 0)),
            pl.BlockSpec((DIM, DIM), lambda i: (0, 0)),
            pl.BlockSpec((1, DIM), lambda i: (0, 0)),
            pl.BlockSpec((1, 1), lambda i: (0, 0), memory_space=pltpu.SMEM),
        ],
        out_specs=pl.BlockSpec((BLK, DIM), lambda i: (i, 0)),
        out_shape=jax.ShapeDtypeStruct((N_NODES, DIM), jnp.float32),
    )(x, agg3d, agg3d, W1, b1.reshape(1, DIM), W2, b2.reshape(1, DIM),
      eps.reshape(1, 1))
    return out


# speedup vs baseline: 4.4254x; 1.0272x over previous
import functools

import jax
import jax.numpy as jnp
from jax import lax
from jax.experimental import pallas as pl
from jax.experimental.pallas import tpu as pltpu
from jax.experimental.pallas import tpu_sc as plsc

N_NODES, N_EDGES, DIM = 10000, 320000, 128

NC, NS = 2, 16
NW = NC * NS
CHUNK = 96
EPT = 10080
E_PAD = NW * EPT
NCHUNK = EPT // CHUNK
N_PAD = 10240
RPT = N_PAD // NS
BLK = 1000


ER = N_EDGES // 128
PR = E_PAD // 128 - ER


def _relu_body(x_ref, o_ref):
    o_ref[...] = jnp.maximum(x_ref[...], 0.0)


def _pad_body(e_ref, sp_ref, dp_ref):
    sp_ref[0:ER] = e_ref[0]
    dp_ref[0:ER] = e_ref[1]
    g = (lax.broadcasted_iota(jnp.int32, (PR, 128), 0) * 128
         + lax.broadcasted_iota(jnp.int32, (PR, 128), 1))
    sp_ref[ER:ER + PR] = g % N_NODES
    dp_ref[ER:ER + PR] = N_NODES + g % (N_PAD - N_NODES)


def _mlp_body(x_ref, a0_ref, a1_ref, w1_ref, b1_ref, w2_ref, b2_ref,
              eps_ref, o_ref):
    h = x_ref[...] * (1.0 + eps_ref[0, 0]) + a0_ref[0] + a1_ref[0]
    h = jnp.dot(h, w1_ref[...], preferred_element_type=jnp.float32)
    h = jnp.maximum(h + b1_ref[...], 0.0)
    o = jnp.dot(h, w2_ref[...], preferred_element_type=jnp.float32)
    o_ref[...] = o + b2_ref[...]


NB = 3
UNROLL = 3
NROUND = NCHUNK // UNROLL


def _sc_agg_body(xr_hbm, src_hbm, dst_hbm, zeros_hbm, agg_hbm, *rest):
    sidx = rest[0:NB]
    didx = rest[NB:2 * NB]
    rows = rest[2 * NB:3 * NB]
    isem = rest[3 * NB:4 * NB]
    gsem = rest[4 * NB:5 * NB]
    acc = rest[5 * NB]

    c = lax.axis_index("c")
    s = lax.axis_index("s")
    wid = s * NC + c
    pltpu.sync_copy(zeros_hbm, acc.at[pl.ds(s * RPT, RPT)])
    plsc.subcore_barrier()

    base = wid * EPT
    last_off = base + (NCHUNK - 1) * CHUNK

    def idx_start(off, b):
        pltpu.make_async_copy(src_hbm.at[pl.ds(off, CHUNK)], sidx[b],
                              isem[b]).start()
        pltpu.make_async_copy(dst_hbm.at[pl.ds(off, CHUNK)], didx[b],
                              isem[b]).start()

    def idx_wait(b):
        pltpu.make_async_copy(src_hbm.at[pl.ds(base, CHUNK)], sidx[b],
                              isem[b]).wait()
        pltpu.make_async_copy(dst_hbm.at[pl.ds(base, CHUNK)], didx[b],
                              isem[b]).wait()

    def g_start(b):
        pltpu.make_async_copy(xr_hbm.at[sidx[b]], rows[b], gsem[b]).start()

    def g_wait(b):
        pltpu.make_async_copy(xr_hbm.at[sidx[b]], rows[b], gsem[b]).wait()

    for j in range(NB):
        idx_start(base + j * CHUNK, j)
    idx_wait(0)
    g_start(0)
    idx_wait(1)
    g_start(1)

    def round_body(r, carry):
        k0 = r * UNROLL
        for u in range(UNROLL):
            k = k0 + u
            s0 = u % NB
            s2 = (u + 2) % NB
            g_wait(s0)
            idx_wait(s2)
            g_start(s2)
            pltpu.sync_copy(rows[s0], acc.at[didx[s0]], add=True)
            off = jnp.minimum(base + (k + NB) * CHUNK, last_off)
            idx_start(off, s0)
        return carry

    lax.fori_loop(0, NROUND, round_body, 0)

    g_wait(0)
    g_wait(1)
    idx_wait(2)

    plsc.subcore_barrier()
    pltpu.sync_copy(acc.at[pl.ds(s * RPT, RPT)],
                    agg_hbm.at[pl.ds(c * N_PAD + s * RPT, RPT)])


def kernel(x, edge_index, W1, b1, W2, b2, eps):
    edge3d = edge_index.reshape(2, ER, 128)
    sp, dp = pl.pallas_call(
        _pad_body,
        out_shape=[
            jax.ShapeDtypeStruct((ER + PR, 128), jnp.int32),
            jax.ShapeDtypeStruct((ER + PR, 128), jnp.int32),
        ],
    )(edge3d)
    src = sp.reshape(E_PAD)
    dst = dp.reshape(E_PAD)

    xr = pl.pallas_call(
        _relu_body,
        grid=(N_NODES // BLK,),
        in_specs=[pl.BlockSpec((BLK, DIM), lambda i: (i, 0))],
        out_specs=pl.BlockSpec((BLK, DIM), lambda i: (i, 0)),
        out_shape=jax.ShapeDtypeStruct((N_NODES, DIM), jnp.float32),
    )(x)

    agg_fn = pl.kernel(
        _sc_agg_body,
        out_type=jax.ShapeDtypeStruct((NC * N_PAD, DIM), jnp.float32),
        mesh=plsc.VectorSubcoreMesh(core_axis_name="c", subcore_axis_name="s"),
        scratch_types=(
            [pltpu.VMEM((CHUNK,), jnp.int32)] * NB
            + [pltpu.VMEM((CHUNK,), jnp.int32)] * NB
            + [pltpu.VMEM((CHUNK, DIM), jnp.float32)] * NB
            + [pltpu.SemaphoreType.DMA] * NB
            + [pltpu.SemaphoreType.DMA] * NB
            + [pltpu.VMEM_SHARED((N_PAD, DIM), jnp.float32)]
        ),
    )
    aggp = agg_fn(xr, src, dst, jnp.zeros((RPT, DIM), jnp.float32))
    agg3d = aggp.reshape(NC, N_PAD, DIM)

    out = pl.pallas_call(
        _mlp_body,
        grid=(N_NODES // BLK,),
        in_specs=[
            pl.BlockSpec((BLK, DIM), lambda i: (i, 0)),
            pl.BlockSpec((1, BLK, DIM), lambda i: (0, i, 0)),
            pl.BlockSpec((1, BLK, DIM), lambda i: (1, i, 0)),
            pl.BlockSpec((DIM, DIM), lambda i: (0, 0)),
            pl.BlockSpec((1, DIM), lambda i: (0, 0)),
            pl.BlockSpec((DIM, DIM), lambda i: (0, 0)),
            pl.BlockSpec((1, DIM), lambda i: (0, 0)),
            pl.BlockSpec((1, 1), lambda i: (0, 0), memory_space=pltpu.SMEM),
        ],
        out_specs=pl.BlockSpec((BLK, DIM), lambda i: (i, 0)),
        out_shape=jax.ShapeDtypeStruct((N_NODES, DIM), jnp.float32),
    )(x, agg3d, agg3d, W1, b1.reshape(1, DIM), W2, b2.reshape(1, DIM),
      eps.reshape(1, 1))
    return out


# speedup vs baseline: 4.5698x; 1.0326x over previous
import functools

import jax
import jax.numpy as jnp
from jax import lax
from jax.experimental import pallas as pl
from jax.experimental.pallas import tpu as pltpu
from jax.experimental.pallas import tpu_sc as plsc

N_NODES, N_EDGES, DIM = 10000, 320000, 128

NC, NS = 2, 16
NW = NC * NS
CHUNK = 96
EPT = 10080
E_PAD = NW * EPT
NCHUNK = EPT // CHUNK
N_PAD = 10240
RPT = N_PAD // NS
BLK = 2000
RBLK = 2000


ER = N_EDGES // 128
PR = E_PAD // 128 - ER


def _relu_body(x_ref, o_ref):
    o_ref[...] = jnp.maximum(x_ref[...], 0.0)


def _pad_body(e_ref, sp_ref, dp_ref):
    sp_ref[0:ER] = e_ref[0]
    dp_ref[0:ER] = e_ref[1]
    g = (lax.broadcasted_iota(jnp.int32, (PR, 128), 0) * 128
         + lax.broadcasted_iota(jnp.int32, (PR, 128), 1))
    sp_ref[ER:ER + PR] = g % N_NODES
    dp_ref[ER:ER + PR] = N_NODES + g % (N_PAD - N_NODES)


def _mlp_body(x_ref, a0_ref, a1_ref, w1_ref, b1_ref, w2_ref, b2_ref,
              eps_ref, o_ref):
    h = x_ref[...] * (1.0 + eps_ref[0, 0]) + a0_ref[0] + a1_ref[0]
    h = jnp.dot(h, w1_ref[...], preferred_element_type=jnp.float32)
    h = jnp.maximum(h + b1_ref[...], 0.0)
    o = jnp.dot(h, w2_ref[...], preferred_element_type=jnp.float32)
    o_ref[...] = o + b2_ref[...]


NB = 3
UNROLL = 3
NROUND = NCHUNK // UNROLL


def _sc_agg_body(xr_hbm, src_hbm, dst_hbm, zeros_hbm, agg_hbm, *rest):
    sidx = rest[0:NB]
    didx = rest[NB:2 * NB]
    rows = rest[2 * NB:3 * NB]
    isem = rest[3 * NB:4 * NB]
    gsem = rest[4 * NB:5 * NB]
    acc = rest[5 * NB]

    c = lax.axis_index("c")
    s = lax.axis_index("s")
    wid = s * NC + c
    base = wid * EPT
    last_off = base + (NCHUNK - 1) * CHUNK

    def idx_start(off, b):
        pltpu.make_async_copy(src_hbm.at[pl.ds(off, CHUNK)], sidx[b],
                              isem[b]).start()
        pltpu.make_async_copy(dst_hbm.at[pl.ds(off, CHUNK)], didx[b],
                              isem[b]).start()

    def idx_wait(b):
        pltpu.make_async_copy(src_hbm.at[pl.ds(base, CHUNK)], sidx[b],
                              isem[b]).wait()
        pltpu.make_async_copy(dst_hbm.at[pl.ds(base, CHUNK)], didx[b],
                              isem[b]).wait()

    def g_start(b):
        pltpu.make_async_copy(xr_hbm.at[sidx[b]], rows[b], gsem[b]).start()

    def g_wait(b):
        pltpu.make_async_copy(xr_hbm.at[sidx[b]], rows[b], gsem[b]).wait()

    for j in range(NB):
        idx_start(base + j * CHUNK, j)
    idx_wait(0)
    g_start(0)
    idx_wait(1)
    g_start(1)
    pltpu.sync_copy(zeros_hbm, acc.at[pl.ds(s * RPT, RPT)])
    plsc.subcore_barrier()

    def round_body(r, carry):
        k0 = r * UNROLL
        for u in range(UNROLL):
            k = k0 + u
            s0 = u % NB
            s2 = (u + 2) % NB
            g_wait(s0)
            idx_wait(s2)
            g_start(s2)
            pltpu.sync_copy(rows[s0], acc.at[didx[s0]], add=True)
            off = jnp.minimum(base + (k + NB) * CHUNK, last_off)
            idx_start(off, s0)
        return carry

    lax.fori_loop(0, NROUND, round_body, 0)

    g_wait(0)
    g_wait(1)
    idx_wait(2)

    plsc.subcore_barrier()
    pltpu.sync_copy(acc.at[pl.ds(s * RPT, RPT)],
                    agg_hbm.at[pl.ds(c * N_PAD + s * RPT, RPT)])


def kernel(x, edge_index, W1, b1, W2, b2, eps):
    edge3d = edge_index.reshape(2, ER, 128)
    sp, dp = pl.pallas_call(
        _pad_body,
        out_shape=[
            jax.ShapeDtypeStruct((ER + PR, 128), jnp.int32),
            jax.ShapeDtypeStruct((ER + PR, 128), jnp.int32),
        ],
    )(edge3d)
    src = sp.reshape(E_PAD)
    dst = dp.reshape(E_PAD)

    xr = pl.pallas_call(
        _relu_body,
        grid=(N_NODES // RBLK,),
        in_specs=[pl.BlockSpec((RBLK, DIM), lambda i: (i, 0))],
        out_specs=pl.BlockSpec((RBLK, DIM), lambda i: (i, 0)),
        out_shape=jax.ShapeDtypeStruct((N_NODES, DIM), jnp.float32),
    )(x)

    agg_fn = pl.kernel(
        _sc_agg_body,
        out_type=jax.ShapeDtypeStruct((NC * N_PAD, DIM), jnp.float32),
        mesh=plsc.VectorSubcoreMesh(core_axis_name="c", subcore_axis_name="s"),
        scratch_types=(
            [pltpu.VMEM((CHUNK,), jnp.int32)] * NB
            + [pltpu.VMEM((CHUNK,), jnp.int32)] * NB
            + [pltpu.VMEM((CHUNK, DIM), jnp.float32)] * NB
            + [pltpu.SemaphoreType.DMA] * NB
            + [pltpu.SemaphoreType.DMA] * NB
            + [pltpu.VMEM_SHARED((N_PAD, DIM), jnp.float32)]
        ),
    )
    aggp = agg_fn(xr, src, dst, jnp.zeros((RPT, DIM), jnp.float32))
    agg3d = aggp.reshape(NC, N_PAD, DIM)

    out = pl.pallas_call(
        _mlp_body,
        grid=(N_NODES // BLK,),
        in_specs=[
            pl.BlockSpec((BLK, DIM), lambda i: (i, 0)),
            pl.BlockSpec((1, BLK, DIM), lambda i: (0, i, 0)),
            pl.BlockSpec((1, BLK, DIM), lambda i: (1, i, 0)),
            pl.BlockSpec((DIM, DIM), lambda i: (0, 0)),
            pl.BlockSpec((1, DIM), lambda i: (0, 0)),
            pl.BlockSpec((DIM, DIM), lambda i: (0, 0)),
            pl.BlockSpec((1, DIM), lambda i: (0, 0)),
            pl.BlockSpec((1, 1), lambda i: (0, 0), memory_space=pltpu.SMEM),
        ],
        out_specs=pl.BlockSpec((BLK, DIM), lambda i: (i, 0)),
        out_shape=jax.ShapeDtypeStruct((N_NODES, DIM), jnp.float32),
    )(x, agg3d, agg3d, W1, b1.reshape(1, DIM), W2, b2.reshape(1, DIM),
      eps.reshape(1, 1))
    return out


# speedup vs baseline: 4.7136x; 1.0315x over previous
import functools

import jax
import jax.numpy as jnp
from jax import lax
from jax.experimental import pallas as pl
from jax.experimental.pallas import tpu as pltpu
from jax.experimental.pallas import tpu_sc as plsc

N_NODES, N_EDGES, DIM = 10000, 320000, 128

NC, NS = 2, 16
NW = NC * NS
CHUNK = 112
EPT = 10080
E_PAD = NW * EPT
NCHUNK = EPT // CHUNK
N_PAD = 10240
RPT = N_PAD // NS
BLK = 2000
RBLK = 2000


ER = N_EDGES // 128
PR = E_PAD // 128 - ER


def _relu_body(x_ref, o_ref):
    o_ref[...] = jnp.maximum(x_ref[...], 0.0)


def _pad_body(e_ref, sp_ref, dp_ref):
    sp_ref[0:ER] = e_ref[0]
    dp_ref[0:ER] = e_ref[1]
    g = (lax.broadcasted_iota(jnp.int32, (PR, 128), 0) * 128
         + lax.broadcasted_iota(jnp.int32, (PR, 128), 1))
    sp_ref[ER:ER + PR] = g % N_NODES
    dp_ref[ER:ER + PR] = N_NODES + g % (N_PAD - N_NODES)


def _mlp_body(x_ref, a0_ref, a1_ref, w1_ref, b1_ref, w2_ref, b2_ref,
              eps_ref, o_ref):
    h = x_ref[...] * (1.0 + eps_ref[0, 0]) + a0_ref[0] + a1_ref[0]
    h = jnp.dot(h, w1_ref[...], preferred_element_type=jnp.float32)
    h = jnp.maximum(h + b1_ref[...], 0.0)
    o = jnp.dot(h, w2_ref[...], preferred_element_type=jnp.float32)
    o_ref[...] = o + b2_ref[...]


NB = 3
UNROLL = 3
NROUND = NCHUNK // UNROLL


def _sc_agg_body(xr_hbm, src_hbm, dst_hbm, zeros_hbm, agg_hbm, *rest):
    sidx = rest[0:NB]
    didx = rest[NB:2 * NB]
    rows = rest[2 * NB:3 * NB]
    isem = rest[3 * NB:4 * NB]
    gsem = rest[4 * NB:5 * NB]
    acc = rest[5 * NB]

    c = lax.axis_index("c")
    s = lax.axis_index("s")
    wid = s * NC + c
    base = wid * EPT
    last_off = base + (NCHUNK - 1) * CHUNK

    def idx_start(off, b):
        pltpu.make_async_copy(src_hbm.at[pl.ds(off, CHUNK)], sidx[b],
                              isem[b]).start()
        pltpu.make_async_copy(dst_hbm.at[pl.ds(off, CHUNK)], didx[b],
                              isem[b]).start()

    def idx_wait(b):
        pltpu.make_async_copy(src_hbm.at[pl.ds(base, CHUNK)], sidx[b],
                              isem[b]).wait()
        pltpu.make_async_copy(dst_hbm.at[pl.ds(base, CHUNK)], didx[b],
                              isem[b]).wait()

    def g_start(b):
        pltpu.make_async_copy(xr_hbm.at[sidx[b]], rows[b], gsem[b]).start()

    def g_wait(b):
        pltpu.make_async_copy(xr_hbm.at[sidx[b]], rows[b], gsem[b]).wait()

    for j in range(NB):
        idx_start(base + j * CHUNK, j)
    idx_wait(0)
    g_start(0)
    idx_wait(1)
    g_start(1)
    pltpu.sync_copy(zeros_hbm, acc.at[pl.ds(s * RPT, RPT)])
    plsc.subcore_barrier()

    def round_body(r, carry):
        k0 = r * UNROLL
        for u in range(UNROLL):
            k = k0 + u
            s0 = u % NB
            s2 = (u + 2) % NB
            g_wait(s0)
            idx_wait(s2)
            g_start(s2)
            pltpu.sync_copy(rows[s0], acc.at[didx[s0]], add=True)
            off = jnp.minimum(base + (k + NB) * CHUNK, last_off)
            idx_start(off, s0)
        return carry

    lax.fori_loop(0, NROUND, round_body, 0)

    g_wait(0)
    g_wait(1)
    idx_wait(2)

    plsc.subcore_barrier()
    pltpu.sync_copy(acc.at[pl.ds(s * RPT, RPT)],
                    agg_hbm.at[pl.ds(c * N_PAD + s * RPT, RPT)])


def kernel(x, edge_index, W1, b1, W2, b2, eps):
    edge3d = edge_index.reshape(2, ER, 128)
    sp, dp = pl.pallas_call(
        _pad_body,
        out_shape=[
            jax.ShapeDtypeStruct((ER + PR, 128), jnp.int32),
            jax.ShapeDtypeStruct((ER + PR, 128), jnp.int32),
        ],
    )(edge3d)
    src = sp.reshape(E_PAD)
    dst = dp.reshape(E_PAD)

    xr = pl.pallas_call(
        _relu_body,
        grid=(N_NODES // RBLK,),
        in_specs=[pl.BlockSpec((RBLK, DIM), lambda i: (i, 0))],
        out_specs=pl.BlockSpec((RBLK, DIM), lambda i: (i, 0)),
        out_shape=jax.ShapeDtypeStruct((N_NODES, DIM), jnp.float32),
    )(x)

    agg_fn = pl.kernel(
        _sc_agg_body,
        out_type=jax.ShapeDtypeStruct((NC * N_PAD, DIM), jnp.float32),
        mesh=plsc.VectorSubcoreMesh(core_axis_name="c", subcore_axis_name="s"),
        scratch_types=(
            [pltpu.VMEM((CHUNK,), jnp.int32)] * NB
            + [pltpu.VMEM((CHUNK,), jnp.int32)] * NB
            + [pltpu.VMEM((CHUNK, DIM), jnp.float32)] * NB
            + [pltpu.SemaphoreType.DMA] * NB
            + [pltpu.SemaphoreType.DMA] * NB
            + [pltpu.VMEM_SHARED((N_PAD, DIM), jnp.float32)]
        ),
    )
    aggp = agg_fn(xr, src, dst, jnp.zeros((RPT, DIM), jnp.float32))
    agg3d = aggp.reshape(NC, N_PAD, DIM)

    out = pl.pallas_call(
        _mlp_body,
        grid=(N_NODES // BLK,),
        in_specs=[
            pl.BlockSpec((BLK, DIM), lambda i: (i, 0)),
            pl.BlockSpec((1, BLK, DIM), lambda i: (0, i, 0)),
            pl.BlockSpec((1, BLK, DIM), lambda i: (1, i, 0)),
            pl.BlockSpec((DIM, DIM), lambda i: (0, 0)),
            pl.BlockSpec((1, DIM), lambda i: (0, 0)),
            pl.BlockSpec((DIM, DIM), lambda i: (0, 0)),
            pl.BlockSpec((1, DIM), lambda i: (0, 0)),
            pl.BlockSpec((1, 1), lambda i: (0, 0), memory_space=pltpu.SMEM),
        ],
        out_specs=pl.BlockSpec((BLK, DIM), lambda i: (i, 0)),
        out_shape=jax.ShapeDtypeStruct((N_NODES, DIM), jnp.float32),
    )(x, agg3d, agg3d, W1, b1.reshape(1, DIM), W2, b2.reshape(1, DIM),
      eps.reshape(1, 1))
    return out
